# Initial kernel scaffold; baseline (speedup 1.0000x reference)
import jax
import jax.numpy as jnp
from jax.experimental import pallas as pl


def kernel(init_user_embedding, init_item_embedding, u_w, i_w, prelu_a, rows0, cols0, vals0, rows1, cols1, vals1, rows2, cols2, vals2):
    raise NotImplementedError("write your pallas kernel here")


# speedup vs baseline: 5.0957x; 5.0957x over previous
import functools

import jax
import jax.numpy as jnp
from jax import lax
from jax.experimental import pallas as pl
from jax.experimental.pallas import tpu as pltpu
from jax.experimental.pallas import tpu_sc as plsc

U = 5000
I = 5000
D = 128
NNZ = 320000
NB = 3

NC = 2
NS = 16
CH = 125
NCH = NNZ // (NS * CH)
EPT = NNZ // NS
RPT = 320
ACC_ROWS = NS * RPT


def _sc_spmm(rows_hbm, cols_hbm, vals0_hbm, vals1_hbm, vals2_hbm,
             utab_hbm, itab_hbm, zeros_hbm,
             ue_out, ie_out, idx_src_v, idx_dst_v, vals_v, gbuf, acc, sem):
    vals_hbm = (vals0_hbm, vals1_hbm, vals2_hbm)
    c = lax.axis_index("c")
    s = lax.axis_index("s")

    def do_side(src_tab, src_idx_hbm, dst_idx_hbm, out_hbm):
        for b in range(NB):
            pltpu.sync_copy(zeros_hbm.at[pl.ds(s * RPT, RPT)],
                            acc.at[pl.ds(s * RPT, RPT)])
            pltpu.sync_copy(src_idx_hbm.at[b, pl.ds(NCH * s, NCH)], idx_src_v)
            pltpu.sync_copy(dst_idx_hbm.at[b, pl.ds(NCH * s, NCH)], idx_dst_v)
            pltpu.sync_copy(vals_hbm[b].at[pl.ds(EPT * s, EPT)],
                            vals_v.at[pl.ds(0, EPT)])
            plsc.subcore_barrier()

            def chunk_body(i, carry):
                pltpu.async_copy(src_tab.at[idx_src_v.at[i]], gbuf, sem).wait()

                def edge_body(e, carry2):
                    val = vals_v[pl.ds(i * CH + e, 16)][0]
                    for j in range(D // 16):
                        gbuf[e, pl.ds(16 * j, 16)] = (
                            gbuf[e, pl.ds(16 * j, 16)] * val)
                    return carry2

                lax.fori_loop(0, CH, edge_body, 0, unroll=False)
                pltpu.sync_copy(gbuf, acc.at[idx_dst_v.at[i]], add=True)
                return carry

            lax.fori_loop(0, NCH, chunk_body, 0, unroll=False)
            plsc.subcore_barrier()

            @pl.when(s < NS - 1)
            def _():
                pltpu.sync_copy(acc.at[pl.ds(s * RPT, RPT)],
                                out_hbm.at[b, pl.ds(s * RPT, RPT)])

            @pl.when(s == NS - 1)
            def _():
                last = U - (NS - 1) * RPT
                pltpu.sync_copy(acc.at[pl.ds((NS - 1) * RPT, last)],
                                out_hbm.at[b, pl.ds((NS - 1) * RPT, last)])

            plsc.subcore_barrier()

    @pl.when(c == 0)
    def _():
        do_side(itab_hbm, cols_hbm, rows_hbm, ue_out)

    @pl.when(c == 1)
    def _():
        do_side(utab_hbm, rows_hbm, cols_hbm, ie_out)


@jax.jit
def _spmm_all(rows_s, cols_s, vals0, vals1, vals2, utab, itab, zeros):
    mesh = plsc.VectorSubcoreMesh(core_axis_name="c", subcore_axis_name="s",
                                  num_cores=NC, num_subcores=NS)
    f = pl.kernel(
        _sc_spmm,
        out_type=(jax.ShapeDtypeStruct((NB, U, D), jnp.float32),
                  jax.ShapeDtypeStruct((NB, I, D), jnp.float32)),
        mesh=mesh,
        scratch_types=[
            pltpu.VMEM((NCH, CH), jnp.int32),
            pltpu.VMEM((NCH, CH), jnp.int32),
            pltpu.VMEM((EPT + 16,), jnp.float32),
            pltpu.VMEM((CH, D), jnp.float32),
            pltpu.VMEM_SHARED((ACC_ROWS, D), jnp.float32),
            pltpu.SemaphoreType.DMA,
        ],
    )
    return f(rows_s, cols_s, vals0, vals1, vals2, utab, itab, zeros)


def _prelu(x, a):
    return jnp.where(x >= 0, x, a * x)


def _tc_body(a_ref, ue_ref, ie_ref, uw_ref, iw_ref,
             mu_ref, mi_ref, su_ref, si_ref):
    a = a_ref[0]
    uw = uw_ref[...]
    iw = iw_ref[...]
    yu = []
    yi = []
    for b in range(NB):
        yu.append(jnp.dot(ue_ref[b], uw, preferred_element_type=jnp.float32))
        yi.append(jnp.dot(ie_ref[b], iw, preferred_element_type=jnp.float32))
        su_ref[b] = _prelu(yu[b], a)
        si_ref[b] = _prelu(yi[b], a)
    third = jnp.float32(1.0 / 3.0)
    mu_ref[...] = _prelu((yu[0] + yu[1] + yu[2]) * third, a)
    mi_ref[...] = _prelu((yi[0] + yi[1] + yi[2]) * third, a)


@jax.jit
def _project(ue, ie, u_w, i_w, prelu_a):
    R = 1000
    grid = (U // R,)
    out_shapes = (
        jax.ShapeDtypeStruct((U, D), jnp.float32),
        jax.ShapeDtypeStruct((I, D), jnp.float32),
        jax.ShapeDtypeStruct((NB, U, D), jnp.float32),
        jax.ShapeDtypeStruct((NB, I, D), jnp.float32),
    )
    return pl.pallas_call(
        _tc_body,
        grid=grid,
        in_specs=[
            pl.BlockSpec(memory_space=pltpu.SMEM),
            pl.BlockSpec((NB, R, D), lambda i: (0, i, 0)),
            pl.BlockSpec((NB, R, D), lambda i: (0, i, 0)),
            pl.BlockSpec((D, D), lambda i: (0, 0)),
            pl.BlockSpec((D, D), lambda i: (0, 0)),
        ],
        out_specs=(
            pl.BlockSpec((R, D), lambda i: (i, 0)),
            pl.BlockSpec((R, D), lambda i: (i, 0)),
            pl.BlockSpec((NB, R, D), lambda i: (0, i, 0)),
            pl.BlockSpec((NB, R, D), lambda i: (0, i, 0)),
        ),
        out_shape=out_shapes,
    )(prelu_a.reshape(1), ue, ie, u_w, i_w)


def kernel(init_user_embedding, init_item_embedding, u_w, i_w, prelu_a,
           rows0, cols0, vals0, rows1, cols1, vals1, rows2, cols2, vals2):
    rows_s = jnp.stack([rows0, rows1, rows2]).reshape(NB, NS * NCH, CH)
    cols_s = jnp.stack([cols0, cols1, cols2]).reshape(NB, NS * NCH, CH)
    zeros = jnp.zeros((ACC_ROWS, D), jnp.float32)
    ue, ie = _spmm_all(rows_s, cols_s, vals0, vals1, vals2,
                       init_user_embedding, init_item_embedding, zeros)
    multi_user, multi_item, single_user, single_item = _project(
        ue, ie, u_w, i_w, prelu_a)
    return (multi_user, multi_item, single_user, single_item)


# speedup vs baseline: 11.3183x; 2.2212x over previous
import jax
import jax.numpy as jnp
from jax import lax
from jax.experimental import pallas as pl
from jax.experimental.pallas import tpu as pltpu
from jax.experimental.pallas import tpu_sc as plsc

U = 5000
I = 5000
D = 128
NNZ = 320000
NB = 3

NC = 2
NS = 16
CH = 80
NCH = NNZ // (NS * CH)
EPT = NNZ // NS
RPT = 320
ACC_ROWS = NS * RPT
NBUF = 5


def _sc_spmm(rows2d_hbm, cols2d_hbm, rowsf_hbm, colsf_hbm, valsf_hbm,
             utab_hbm, itab_hbm, zeros_hbm,
             ue_out, ie_out,
             idx_dst_v, isrc_bufs, val_bufs, gbufs, isems, gsems, ssems, acc):
    c = lax.axis_index("c")
    s = lax.axis_index("s")

    def do_side(src_tab, src_idx_flat, dst_idx_2d, out_hbm):
        for b in range(NB):
            pltpu.sync_copy(zeros_hbm.at[pl.ds(s * RPT, RPT)],
                            acc.at[pl.ds(s * RPT, RPT)])
            pltpu.sync_copy(dst_idx_2d.at[b * NS + s], idx_dst_v)
            plsc.subcore_barrier()
            ebase = s * EPT

            def ifetch_start(i, k):
                off = ebase + i * CH
                pltpu.async_copy(src_idx_flat[b].at[pl.ds(off, CH)],
                                 isrc_bufs[k], isems[k])
                pltpu.async_copy(valsf_hbm[b].at[pl.ds(off, CH)],
                                 val_bufs[k].at[pl.ds(0, CH)], isems[k])

            def ifetch_wait(i, k):
                off = ebase + i * CH
                pltpu.make_async_copy(src_idx_flat[b].at[pl.ds(off, CH)],
                                      isrc_bufs[k], isems[k]).wait()
                pltpu.make_async_copy(valsf_hbm[b].at[pl.ds(off, CH)],
                                      val_bufs[k].at[pl.ds(0, CH)],
                                      isems[k]).wait()

            def gather_start(k):
                pltpu.async_copy(src_tab.at[isrc_bufs[k]], gbufs[k], gsems[k])

            def gather_wait(k):
                pltpu.make_async_copy(src_tab.at[isrc_bufs[k]],
                                      gbufs[k], gsems[k]).wait()

            def scatter_start(i, k):
                pltpu.async_copy(gbufs[k], acc.at[idx_dst_v.at[i]],
                                 ssems[k], add=True)

            def scatter_wait(i, k):
                pltpu.make_async_copy(gbufs[k], acc.at[idx_dst_v.at[i]],
                                      ssems[k]).wait()

            def scale(k):
                buf = gbufs[k]
                vbuf = val_bufs[k]

                def edge_body(e, carry2):
                    val = vbuf[pl.ds(e, 16)][0]
                    for j in range(D // 16):
                        buf[e, pl.ds(16 * j, 16)] = (
                            buf[e, pl.ds(16 * j, 16)] * val)
                    return carry2

                lax.fori_loop(0, CH, edge_body, 0, unroll=5)

            for k in range(3):
                ifetch_start(k, k)
            for k in range(2):
                ifetch_wait(k, k)
                gather_start(k)

            def chunk_group(g, carry):
                for j in range(NBUF):
                    ch = g * NBUF + j
                    pre = ch + 3
                    kpre = (j + 3) % NBUF
                    mid = ch + 2
                    kmid = (j + 2) % NBUF

                    @pl.when(pre < NCH)
                    def _():
                        @pl.when(pre >= NBUF)
                        def _():
                            scatter_wait(pre - NBUF, kpre)

                        ifetch_start(pre, kpre)

                    @pl.when(mid < NCH)
                    def _():
                        ifetch_wait(mid, kmid)
                        gather_start(kmid)

                    gather_wait(j)
                    scale(j)
                    scatter_start(ch, j)
                return carry

            lax.fori_loop(0, NCH // NBUF, chunk_group, 0, unroll=False)
            for j in range(NBUF):
                scatter_wait(NCH - NBUF + j, j)
            plsc.subcore_barrier()

            @pl.when(s < NS - 1)
            def _():
                pltpu.sync_copy(acc.at[pl.ds(s * RPT, RPT)],
                                out_hbm.at[b, pl.ds(s * RPT, RPT)])

            @pl.when(s == NS - 1)
            def _():
                last = U - (NS - 1) * RPT
                pltpu.sync_copy(acc.at[pl.ds((NS - 1) * RPT, last)],
                                out_hbm.at[b, pl.ds((NS - 1) * RPT, last)])

            plsc.subcore_barrier()

    @pl.when(c == 0)
    def _():
        do_side(itab_hbm, colsf_hbm, rows2d_hbm, ue_out)

    @pl.when(c == 1)
    def _():
        do_side(utab_hbm, rowsf_hbm, cols2d_hbm, ie_out)


@jax.jit
def _spmm_all(rows2d, cols2d, rf0, rf1, rf2, cf0, cf1, cf2,
              v0, v1, v2, utab, itab, zeros):
    mesh = plsc.VectorSubcoreMesh(core_axis_name="c", subcore_axis_name="s",
                                  num_cores=NC, num_subcores=NS)
    f = pl.kernel(
        lambda *a: _sc_spmm(a[0], a[1], (a[2], a[3], a[4]), (a[5], a[6], a[7]),
                            (a[8], a[9], a[10]), *a[11:]),
        out_type=(jax.ShapeDtypeStruct((NB, U, D), jnp.float32),
                  jax.ShapeDtypeStruct((NB, I, D), jnp.float32)),
        mesh=mesh,
        scratch_types=[
            pltpu.VMEM((NCH, CH), jnp.int32),
            [pltpu.VMEM((CH,), jnp.int32) for _ in range(NBUF)],
            [pltpu.VMEM((CH + 16,), jnp.float32) for _ in range(NBUF)],
            [pltpu.VMEM((CH, D), jnp.float32) for _ in range(NBUF)],
            [pltpu.SemaphoreType.DMA for _ in range(NBUF)],
            [pltpu.SemaphoreType.DMA for _ in range(NBUF)],
            [pltpu.SemaphoreType.DMA for _ in range(NBUF)],
            pltpu.VMEM_SHARED((ACC_ROWS, D), jnp.float32),
        ],
    )
    return f(rows2d, cols2d, rf0, rf1, rf2, cf0, cf1, cf2,
             v0, v1, v2, utab, itab, zeros)


def _prelu(x, a):
    return jnp.where(x >= 0, x, a * x)


def _tc_body(a_ref, ue_ref, ie_ref, uw_ref, iw_ref,
             mu_ref, mi_ref, su_ref, si_ref):
    a = a_ref[0]
    uw = uw_ref[...]
    iw = iw_ref[...]
    yu = []
    yi = []
    for b in range(NB):
        yu.append(jnp.dot(ue_ref[b], uw, preferred_element_type=jnp.float32))
        yi.append(jnp.dot(ie_ref[b], iw, preferred_element_type=jnp.float32))
        su_ref[b] = _prelu(yu[b], a)
        si_ref[b] = _prelu(yi[b], a)
    third = jnp.float32(1.0 / 3.0)
    mu_ref[...] = _prelu((yu[0] + yu[1] + yu[2]) * third, a)
    mi_ref[...] = _prelu((yi[0] + yi[1] + yi[2]) * third, a)


@jax.jit
def _project(ue, ie, u_w, i_w, prelu_a):
    R = 1000
    grid = (U // R,)
    out_shapes = (
        jax.ShapeDtypeStruct((U, D), jnp.float32),
        jax.ShapeDtypeStruct((I, D), jnp.float32),
        jax.ShapeDtypeStruct((NB, U, D), jnp.float32),
        jax.ShapeDtypeStruct((NB, I, D), jnp.float32),
    )
    return pl.pallas_call(
        _tc_body,
        grid=grid,
        in_specs=[
            pl.BlockSpec(memory_space=pltpu.SMEM),
            pl.BlockSpec((NB, R, D), lambda i: (0, i, 0)),
            pl.BlockSpec((NB, R, D), lambda i: (0, i, 0)),
            pl.BlockSpec((D, D), lambda i: (0, 0)),
            pl.BlockSpec((D, D), lambda i: (0, 0)),
        ],
        out_specs=(
            pl.BlockSpec((R, D), lambda i: (i, 0)),
            pl.BlockSpec((R, D), lambda i: (i, 0)),
            pl.BlockSpec((NB, R, D), lambda i: (0, i, 0)),
            pl.BlockSpec((NB, R, D), lambda i: (0, i, 0)),
        ),
        out_shape=out_shapes,
    )(prelu_a.reshape(1), ue, ie, u_w, i_w)


def kernel(init_user_embedding, init_item_embedding, u_w, i_w, prelu_a,
           rows0, cols0, vals0, rows1, cols1, vals1, rows2, cols2, vals2):
    rows2d = jnp.stack([rows0, rows1, rows2]).reshape(NB * NS, NCH, CH)
    cols2d = jnp.stack([cols0, cols1, cols2]).reshape(NB * NS, NCH, CH)
    zeros = jnp.zeros((ACC_ROWS, D), jnp.float32)
    ue, ie = _spmm_all(rows2d, cols2d, rows0, rows1, rows2,
                       cols0, cols1, cols2, vals0, vals1, vals2,
                       init_user_embedding, init_item_embedding, zeros)
    multi_user, multi_item, single_user, single_item = _project(
        ue, ie, u_w, i_w, prelu_a)
    return (multi_user, multi_item, single_user, single_item)
